# baseline (speedup 1.0000x reference)
import functools

import jax
import jax.numpy as jnp
from jax import lax
from jax.experimental import pallas as pl
from jax.experimental.pallas import tpu as pltpu
from jax.experimental.pallas import tpu_sc as plsc

B = 4
C_IN = 512
C_OUT = 384
D = 8192
NROWS_OUT = B * C_OUT
NW = 32
ROWS_PER_TILE = NROWS_OUT // NW
FLIGHT = 8

_mesh = plsc.VectorSubcoreMesh(core_axis_name="c", subcore_axis_name="s")


@functools.partial(
    pl.kernel,
    mesh=_mesh,
    compiler_params=pltpu.CompilerParams(needs_layout_passes=False),
    out_type=jax.ShapeDtypeStruct((NROWS_OUT, D), jnp.float32),
    scratch_types=[
        pltpu.VMEM((ROWS_PER_TILE,), jnp.int32),
        pltpu.SMEM((ROWS_PER_TILE,), jnp.int32),
        pltpu.SemaphoreType.DMA,
    ],
)
def _sc_gather(table_hbm, fidx_hbm, out_hbm, idx_v, idx_s, sem):
    wid = lax.axis_index("s") * 2 + lax.axis_index("c")
    base = wid * ROWS_PER_TILE
    pltpu.sync_copy(fidx_hbm.at[wid], idx_v)

    lane = lax.iota(jnp.int32, 16)
    copies = []
    for r in range(ROWS_PER_TILE):
        if len(copies) == FLIGHT:
            copies.pop(0).wait()
        chunk = idx_v[pl.ds((r // 16) * 16, 16)]
        row = jnp.sum(jnp.where(lane == (r % 16), chunk, 0))
        copies.append(pltpu.async_copy(
            table_hbm.at[row], out_hbm.at[base + r], sem))
    for cp in copies:
        cp.wait()


def kernel(x, indices):
    table = x.reshape(B * C_IN, D)
    fidx = jnp.arange(B, dtype=jnp.int32)[:, None] * C_IN + indices[None, :]
    fidx = fidx.reshape(NW, ROWS_PER_TILE)
    out = _sc_gather(table, fidx)
    return out.reshape(B, C_OUT, D)


# speedup vs baseline: 27.2196x; 27.2196x over previous
import functools

import jax
import jax.numpy as jnp
from jax import lax
from jax.experimental import pallas as pl
from jax.experimental.pallas import tpu as pltpu
from jax.experimental.pallas import tpu_sc as plsc

B = 4
C_IN = 512
C_OUT = 384
D = 8192
NROWS_OUT = B * C_OUT
NW = 32
ROWS_PER_TILE = NROWS_OUT // NW
CHUNK = 4
NCHUNK = ROWS_PER_TILE // CHUNK
NBUF = 3

_mesh = plsc.VectorSubcoreMesh(core_axis_name="c", subcore_axis_name="s")


@functools.partial(
    pl.kernel,
    mesh=_mesh,
    out_type=jax.ShapeDtypeStruct((NROWS_OUT, D), jnp.float32),
    scratch_types=[
        pltpu.VMEM((NCHUNK, CHUNK), jnp.int32),
        *[pltpu.VMEM((CHUNK, D), jnp.float32) for _ in range(NBUF)],
        *[pltpu.SemaphoreType.DMA for _ in range(2 * NBUF)],
    ],
)
def _sc_gather(table_hbm, fidx_hbm, out_hbm, idx_v, *bufs_and_sems):
    bufs = bufs_and_sems[:NBUF]
    gsems = bufs_and_sems[NBUF:2 * NBUF]
    ssems = bufs_and_sems[2 * NBUF:]
    wid = lax.axis_index("s") * 2 + lax.axis_index("c")
    base = wid * ROWS_PER_TILE
    pltpu.sync_copy(fidx_hbm.at[wid], idx_v)

    gathers = [None] * NBUF
    scatters = [None] * NBUF

    gathers[0] = pltpu.async_copy(table_hbm.at[idx_v.at[0]], bufs[0], gsems[0])
    for c in range(NCHUNK):
        nxt = c + 1
        if nxt < NCHUNK:
            sn = nxt % NBUF
            if scatters[sn] is not None:
                scatters[sn].wait()
                scatters[sn] = None
            gathers[sn] = pltpu.async_copy(
                table_hbm.at[idx_v.at[nxt]], bufs[sn], gsems[sn])
        s = c % NBUF
        gathers[s].wait()
        scatters[s] = pltpu.async_copy(
            bufs[s], out_hbm.at[pl.ds(base + c * CHUNK, CHUNK)], ssems[s])
    for s in range(NBUF):
        if scatters[s] is not None:
            scatters[s].wait()


def kernel(x, indices):
    table = x.reshape(B * C_IN, D)
    fidx = jnp.arange(B, dtype=jnp.int32)[:, None] * C_IN + indices[None, :]
    fidx = fidx.reshape(NW, NCHUNK, CHUNK)
    out = _sc_gather(table, fidx)
    return out.reshape(B, C_OUT, D)


# speedup vs baseline: 27.2465x; 1.0010x over previous
import functools

import jax
import jax.numpy as jnp
from jax import lax
from jax.experimental import pallas as pl
from jax.experimental.pallas import tpu as pltpu
from jax.experimental.pallas import tpu_sc as plsc

B = 4
C_IN = 512
C_OUT = 384
D = 8192
NROWS_OUT = B * C_OUT
NW = 32
ROWS_PER_TILE = NROWS_OUT // NW
CHUNK = 4
NCHUNK = ROWS_PER_TILE // CHUNK
NBUF = 3

_mesh = plsc.VectorSubcoreMesh(core_axis_name="c", subcore_axis_name="s")


@functools.partial(
    pl.kernel,
    mesh=_mesh,
    compiler_params=pltpu.CompilerParams(
        skip_device_barrier=True,
        disable_bounds_checks=True,
        disable_semaphore_checks=True,
    ),
    out_type=jax.ShapeDtypeStruct((NROWS_OUT, D), jnp.float32),
    scratch_types=[
        pltpu.VMEM((NCHUNK, CHUNK), jnp.int32),
        *[pltpu.VMEM((CHUNK, D), jnp.float32) for _ in range(NBUF)],
        *[pltpu.SemaphoreType.DMA for _ in range(2 * NBUF)],
    ],
)
def _sc_gather(table_hbm, fidx_hbm, out_hbm, idx_v, *bufs_and_sems):
    bufs = bufs_and_sems[:NBUF]
    gsems = bufs_and_sems[NBUF:2 * NBUF]
    ssems = bufs_and_sems[2 * NBUF:]
    wid = lax.axis_index("s") * 2 + lax.axis_index("c")
    base = wid * ROWS_PER_TILE
    pltpu.sync_copy(fidx_hbm.at[wid], idx_v)

    gathers = [None] * NBUF
    scatters = [None] * NBUF

    gathers[0] = pltpu.async_copy(table_hbm.at[idx_v.at[0]], bufs[0], gsems[0])
    for c in range(NCHUNK):
        nxt = c + 1
        if nxt < NCHUNK:
            sn = nxt % NBUF
            if scatters[sn] is not None:
                scatters[sn].wait()
                scatters[sn] = None
            gathers[sn] = pltpu.async_copy(
                table_hbm.at[idx_v.at[nxt]], bufs[sn], gsems[sn])
        s = c % NBUF
        gathers[s].wait()
        scatters[s] = pltpu.async_copy(
            bufs[s], out_hbm.at[pl.ds(base + c * CHUNK, CHUNK)], ssems[s])
    for s in range(NBUF):
        if scatters[s] is not None:
            scatters[s].wait()


def kernel(x, indices):
    table = x.reshape(B * C_IN, D)
    fidx = jnp.arange(B, dtype=jnp.int32)[:, None] * C_IN + indices[None, :]
    fidx = fidx.reshape(NW, NCHUNK, CHUNK)
    out = _sc_gather(table, fidx)
    return out.reshape(B, C_OUT, D)
